# Initial kernel scaffold; baseline (speedup 1.0000x reference)
import jax
import jax.numpy as jnp
from jax.experimental import pallas as pl


def kernel(index, label, sentence_mask, features, edges, W1, b1, W2, b2, Wfc, bfc):
    raise NotImplementedError("write your pallas kernel here")


# speedup vs baseline: 12.2005x; 12.2005x over previous
import functools

import jax
import jax.numpy as jnp
from jax import lax
from jax.experimental import pallas as pl
from jax.experimental.pallas import tpu as pltpu
from jax.experimental.pallas import tpu_sc as plsc

N = 10000
E = 320000
D = 128
H = 128
C = 64

NC, NS = 2, 16
NW = NC * NS
K = 80
CPT = E // (NW * K)
NP = 10240
RPT = NP // NS
G = 5
NSUP = CPT // G
K2 = 40
CPT2 = E // (NW * K2)
NSUP2 = CPT2 // G


def _mesh():
    return plsc.VectorSubcoreMesh(core_axis_name="c", subcore_axis_name="s")


def _deg_body(i0r_hbm, ones_hbm, zeros_hbm, cnt_hbm, acc, idx_v, ones_v, sem):
    c = lax.axis_index("c")
    s = lax.axis_index("s")
    wid = c * NS + s
    pltpu.sync_copy(ones_hbm, ones_v)
    pltpu.sync_copy(zeros_hbm.at[pl.ds(s * RPT, RPT)], acc.at[pl.ds(s * RPT, RPT)])
    pltpu.sync_copy(i0r_hbm.at[wid], idx_v)
    plsc.subcore_barrier()

    def step(ss, carry):
        descs = [
            pltpu.async_copy(ones_v, acc.at[idx_v.at[ss * NSUP + j]], sem, add=True)
            for j in range(NSUP)
        ]
        for d in descs:
            d.wait()
        return carry

    lax.fori_loop(0, CPT // NSUP, step, 0)
    plsc.subcore_barrier()
    pltpu.sync_copy(acc.at[pl.ds(s * RPT, RPT)],
                    cnt_hbm.at[pl.ds(c * NP + s * RPT, RPT)])


def _deg_call(i0r, ones16, zeros16):
    fn = pl.kernel(
        _deg_body,
        out_type=jax.ShapeDtypeStruct((NC * NP, 16), jnp.float32),
        mesh=_mesh(),
        compiler_params=pltpu.CompilerParams(use_tc_tiling_on_sc=False),
        scratch_types=[
            pltpu.VMEM_SHARED((NP, 16), jnp.float32),
            pltpu.VMEM((CPT, K), jnp.int32),
            pltpu.VMEM((K, 16), jnp.float32),
            pltpu.SemaphoreType.DMA,
        ],
    )
    return fn(i0r, ones16, zeros16)


def _spmm_body(lin_hbm, i0r_hbm, i1r_hbm, zeros_hbm, out_hbm,
               acc, i0_v, i1_v, rows, gsem, ssem):
    c = lax.axis_index("c")
    s = lax.axis_index("s")
    wid = c * NS + s
    pltpu.sync_copy(zeros_hbm.at[pl.ds(s * RPT, RPT)], acc.at[pl.ds(s * RPT, RPT)])
    pltpu.sync_copy(i0r_hbm.at[wid], i0_v)
    pltpu.sync_copy(i1r_hbm.at[wid], i1_v)
    plsc.subcore_barrier()

    def step(ss, carry):
        gd = [
            pltpu.async_copy(lin_hbm.at[i0_v.at[ss * G + g]], rows.at[g], gsem)
            for g in range(G)
        ]
        sd = []
        for g in range(G):
            gd[g].wait()
            sd.append(
                pltpu.async_copy(rows.at[g], acc.at[i1_v.at[ss * G + g]], ssem,
                                 add=True))
        for d in sd:
            d.wait()
        return carry

    lax.fori_loop(0, NSUP2, step, 0)
    plsc.subcore_barrier()
    pltpu.sync_copy(acc.at[pl.ds(s * RPT, RPT)],
                    out_hbm.at[pl.ds(c * NP + s * RPT, RPT)])


def _spmm_call(lin, i0r, i1r, zerosNH):
    fn = pl.kernel(
        _spmm_body,
        out_type=jax.ShapeDtypeStruct((NC * NP, H), jnp.float32),
        mesh=_mesh(),
        compiler_params=pltpu.CompilerParams(use_tc_tiling_on_sc=False),
        scratch_types=[
            pltpu.VMEM_SHARED((NP, H), jnp.float32),
            pltpu.VMEM((CPT2, K2), jnp.int32),
            pltpu.VMEM((CPT2, K2), jnp.int32),
            pltpu.VMEM((G, K2, H), jnp.float32),
            pltpu.SemaphoreType.DMA,
            pltpu.SemaphoreType.DMA,
        ],
    )
    return fn(lin, i0r, i1r, zerosNH)


def _edge_body(p0_hbm, p1_hbm, i0r_hbm, i1r_hbm, x_hbm, i0_v, i1_v, buf, gsem, wsem):
    c = lax.axis_index("c")
    s = lax.axis_index("s")
    wid = c * NS + s
    r0 = wid * CPT
    pltpu.sync_copy(i0r_hbm.at[wid], i0_v)
    pltpu.sync_copy(i1r_hbm.at[wid], i1_v)

    def step(ss, carry):
        gd = [
            pltpu.async_copy(p0_hbm.at[i0_v.at[ss * G + g]], buf.at[g], gsem)
            for g in range(G)
        ]
        for d in gd:
            d.wait()
        ad = [
            pltpu.async_copy(p1_hbm.at[i1_v.at[ss * G + g]], buf.at[g], gsem,
                             add=True)
            for g in range(G)
        ]
        for d in ad:
            d.wait()
        wd = [
            pltpu.async_copy(buf.at[g],
                             x_hbm.at[pl.ds((r0 + ss * G + g) * K, K)], wsem)
            for g in range(G)
        ]
        for d in wd:
            d.wait()
        return carry

    lax.fori_loop(0, NSUP, step, 0)


def _edge_call(p0, p1, i0r, i1r):
    fn = pl.kernel(
        _edge_body,
        out_type=jax.ShapeDtypeStruct((E, C), jnp.float32),
        mesh=_mesh(),
        compiler_params=pltpu.CompilerParams(use_tc_tiling_on_sc=False),
        scratch_types=[
            pltpu.VMEM((CPT, K), jnp.int32),
            pltpu.VMEM((CPT, K), jnp.int32),
            pltpu.VMEM((G, K, C), jnp.float32),
            pltpu.SemaphoreType.DMA,
            pltpu.SemaphoreType.DMA,
        ],
    )
    return fn(p0, p1, i0r, i1r)


def _matT(x, w):
    return lax.dot_general(x, w, (((1,), (1,)), ((), ())),
                           preferred_element_type=jnp.float32)


def _pre_body(cnt_ref, feat_ref, w1_ref, b1_ref, dis_ref, lin_ref):
    deg = cnt_ref[:N, 0:1] + cnt_ref[NP:NP + N, 0:1] + 1.0
    dis = lax.rsqrt(deg)
    dis_ref[...] = dis
    lin_ref[...] = dis * (_matT(feat_ref[...], w1_ref[...]) + b1_ref[...])


def _mid_body(acc_ref, lin1_ref, dis_ref, w2_ref, b2_ref, lin2_ref):
    dis = dis_ref[...]
    h1 = dis * (acc_ref[:N, :] + acc_ref[NP:NP + N, :] + lin1_ref[...])
    lin2_ref[...] = dis * (_matT(h1, w2_ref[...]) + b2_ref[...])


def _post_body(acc_ref, lin2_ref, dis_ref, wfc0_ref, wfc1_ref, bfc_ref,
               p0_ref, p1_ref):
    dis = dis_ref[...]
    h2 = dis * (acc_ref[:N, :] + acc_ref[NP:NP + N, :] + lin2_ref[...])
    p0_ref[...] = _matT(h2, wfc0_ref[...]) + bfc_ref[...]
    p1_ref[...] = _matT(h2, wfc1_ref[...])


_BE = 8000


def _loss_body(x_ref, lab_ref, out_ref):
    i = pl.program_id(0)
    x = x_ref[...]
    lab = lab_ref[...]
    m = jnp.max(x, axis=1, keepdims=True)
    lse = jnp.log(jnp.sum(jnp.exp(x - m), axis=1, keepdims=True)) + m
    onehot = lax.broadcasted_iota(jnp.int32, x.shape, 1) == lab
    xl = jnp.sum(jnp.where(onehot, x, 0.0), axis=1, keepdims=True)
    part = jnp.sum(lse - xl)

    @pl.when(i == 0)
    def _():
        out_ref[...] = jnp.zeros_like(out_ref)

    out_ref[...] += jnp.reshape(part, (1, 1))


def kernel(index, label, sentence_mask, features, edges, W1, b1, W2, b2, Wfc, bfc):
    idx0 = index[0].astype(jnp.int32)
    idx1 = index[1].astype(jnp.int32)
    i0r = idx0.reshape(NW, CPT, K)
    i1r = idx1.reshape(NW, CPT, K)
    i0s = idx0.reshape(NW, CPT2, K2)
    i1s = idx1.reshape(NW, CPT2, K2)
    ones16 = jnp.ones((K, 16), jnp.float32)
    zeros16 = jnp.zeros((NP, 16), jnp.float32)
    zerosNH = jnp.zeros((NP, H), jnp.float32)

    cnt = _deg_call(i0r, ones16, zeros16)

    dis, lin1 = pl.pallas_call(
        _pre_body,
        out_shape=[jax.ShapeDtypeStruct((N, 1), jnp.float32),
                   jax.ShapeDtypeStruct((N, H), jnp.float32)],
    )(cnt, features, W1, b1.reshape(1, H))

    acc1 = _spmm_call(lin1, i0s, i1s, zerosNH)

    lin2 = pl.pallas_call(
        _mid_body,
        out_shape=jax.ShapeDtypeStruct((N, H), jnp.float32),
    )(acc1, lin1, dis, W2, b2.reshape(1, H))

    acc2 = _spmm_call(lin2, i0s, i1s, zerosNH)

    p0, p1 = pl.pallas_call(
        _post_body,
        out_shape=[jax.ShapeDtypeStruct((N, C), jnp.float32),
                   jax.ShapeDtypeStruct((N, C), jnp.float32)],
    )(acc2, lin2, dis, Wfc[:, :H], Wfc[:, H:], bfc.reshape(1, C))

    x = _edge_call(p0, p1, i0r, i1r)

    tot = pl.pallas_call(
        _loss_body,
        grid=(E // _BE,),
        in_specs=[pl.BlockSpec((_BE, C), lambda i: (i, 0)),
                  pl.BlockSpec((_BE, 1), lambda i: (i, 0))],
        out_specs=pl.BlockSpec((1, 1), lambda i: (0, 0)),
        out_shape=jax.ShapeDtypeStruct((1, 1), jnp.float32),
    )(x, label.astype(jnp.int32).reshape(E, 1))

    loss = tot[0, 0] / jnp.float32(E)
    return (loss, x)
